# baseline (speedup 1.0000x reference)
import functools
import jax
import jax.numpy as jnp
from jax import lax
from jax.experimental import pallas as pl
from jax.experimental.pallas import tpu as pltpu
from jax.experimental.pallas import tpu_sc as plsc

NV = 40962
NF = 81920
B = 2
C = 32
D = B * C

NC = 2
NS = 16
NW = NC * NS

FW = NF // NW
CF = 64
NCHA = FW // CF
RA = CF * 9
UA = 72
NUA = RA // UA

NVP = 43008
VW = NVP // NW
CV = 32
NCHV = VW // CV
RL = CV * 7
UL = 28
NUL = RL // UL
RF = CV * 6
UF = 24
NUF = RF // UF

_mesh = plsc.VectorSubcoreMesh(core_axis_name="c", subcore_axis_name="s")


def _wid():
    return lax.axis_index("s") * NC + lax.axis_index("c")


@functools.partial(
    pl.kernel,
    out_type=[
        jax.ShapeDtypeStruct((NF, D), jnp.float32),
        jax.ShapeDtypeStruct((NF, D), jnp.float32),
    ],
    mesh=_mesh,
    compiler_params=pltpu.CompilerParams(use_tc_tiling_on_sc=False),
    scratch_types=(
        [pltpu.VMEM((NUA, UA), jnp.int32)] * 2 +
        [pltpu.VMEM((CF, 16), jnp.float32)] * 6 +
        [pltpu.VMEM((CF, 16), jnp.float32)] * 2 +
        [pltpu.VMEM((RA, D), jnp.float32)] * 2 +
        [pltpu.VMEM((CF, D), jnp.float32)] * 4 +
        [pltpu.SemaphoreType.DMA] * 6
    ),
)
def _faces_kernel(y_hbm, gc_hbm, gv_hbm, ew_hbm, ns_hbm, ewf_hbm, nsf_hbm,
                  idx0, idx1, gv0, gv1, ew0, ew1, ns0, ns1, we_v, wn_v,
                  rows0, rows1, oew0, oew1, ons0, ons1,
                  s_st0, s_st1, s_rw0, s_rw1, s_out0, s_out1):
    wid = _wid()
    slot = (
        dict(idx=idx0, gv=gv0, ew=ew0, ns=ns0, rows=rows0, oew=oew0,
             ons=ons0, s_st=s_st0, s_rw=s_rw0, s_out=s_out0),
        dict(idx=idx1, gv=gv1, ew=ew1, ns=ns1, rows=rows1, oew=oew1,
             ons=ons1, s_st=s_st1, s_rw=s_rw1, s_out=s_out1),
    )

    def stage(c, s):
        d = slot[s]
        face0 = pl.multiple_of(wid * FW + c * CF, 8)
        ubase = wid * (FW * 9 // UA) + c * NUA
        pltpu.async_copy(gc_hbm.at[pl.ds(ubase, NUA)], d["idx"], d["s_st"])
        pltpu.async_copy(gv_hbm.at[pl.ds(face0, CF)], d["gv"], d["s_st"])
        pltpu.async_copy(ew_hbm.at[pl.ds(face0, CF)], d["ew"], d["s_st"])
        pltpu.async_copy(ns_hbm.at[pl.ds(face0, CF)], d["ns"], d["s_st"])

    def wait_stage(s):
        d = slot[s]
        pltpu.make_async_copy(gc_hbm.at[pl.ds(0, NUA)], d["idx"],
                              d["s_st"]).wait()
        pltpu.make_async_copy(gv_hbm.at[pl.ds(0, CF)], d["gv"],
                              d["s_st"]).wait()
        pltpu.make_async_copy(ew_hbm.at[pl.ds(0, CF)], d["ew"],
                              d["s_st"]).wait()
        pltpu.make_async_copy(ns_hbm.at[pl.ds(0, CF)], d["ns"],
                              d["s_st"]).wait()

    def p1(c, s):
        d = slot[s]
        wait_stage(s)
        for u in range(NUA):
            pltpu.async_copy(y_hbm.at[d["idx"].at[u]],
                             d["rows"].at[pl.ds(u * UA, UA)], d["s_rw"])

    def wait_out(s):
        d = slot[s]
        pltpu.make_async_copy(d["oew"], ewf_hbm.at[pl.ds(0, CF)],
                              d["s_out"]).wait()
        pltpu.make_async_copy(d["ons"], nsf_hbm.at[pl.ds(0, CF)],
                              d["s_out"]).wait()

    def p2(c, s, first=False, last=False):
        d = slot[s]
        for u in range(NUA):
            pltpu.make_async_copy(y_hbm.at[d["idx"].at[u]],
                                  d["rows"].at[pl.ds(u * UA, UA)],
                                  d["s_rw"]).wait()
        if not first:
            wait_out(s)

        def combine(t, carry):
            g = d["gv"][t]
            we_v[t] = d["ew"][t] * g
            wn_v[t] = d["ns"][t] * g
            return carry

        lax.fori_loop(0, CF, combine, 0)
        if not last:
            stage(c + 2, s)

        rows_v, oew_v, ons_v = d["rows"], d["oew"], d["ons"]

        def face(i, carry):
            we = we_v[i]
            wn = wn_v[i]
            base = i * 9
            acc = [jnp.zeros((16,), jnp.float32) for _ in range(8)]
            for k in range(9):
                r = base + k
                wek = we[k]
                wnk = wn[k]
                for j in range(4):
                    row = rows_v[r, pl.ds(j * 16, 16)]
                    acc[j] = acc[j] + wek * row
                    acc[4 + j] = acc[4 + j] + wnk * row
            for j in range(4):
                oew_v[i, pl.ds(j * 16, 16)] = acc[j]
                ons_v[i, pl.ds(j * 16, 16)] = acc[4 + j]
            return carry

        lax.fori_loop(0, CF, face, 0)
        face0 = pl.multiple_of(wid * FW + c * CF, 8)
        pltpu.async_copy(oew_v, ewf_hbm.at[pl.ds(face0, CF)], d["s_out"])
        pltpu.async_copy(ons_v, nsf_hbm.at[pl.ds(face0, CF)], d["s_out"])

    stage(0, 0)
    stage(1, 1)
    p1(0, 0)
    p1(1, 1)
    p2(0, 0, first=True)
    p1(2, 0)
    p2(1, 1, first=True)

    def body(t, carry):
        c = 2 * t
        p1(c + 1, 1)
        p2(c, 0)
        p1(c + 2, 0)
        p2(c + 1, 1)
        return carry

    lax.fori_loop(1, NCHA // 2 - 1, body, 0)
    c = NCHA - 2
    p1(c + 1, 1)
    p2(c, 0, last=True)
    p2(c + 1, 1, last=True)
    wait_out(0)
    wait_out(1)


@functools.partial(
    pl.kernel,
    out_type=[
        jax.ShapeDtypeStruct((NVP, D), jnp.float32),
        jax.ShapeDtypeStruct((NVP, D), jnp.float32),
        jax.ShapeDtypeStruct((NVP, D), jnp.float32),
    ],
    mesh=_mesh,
    compiler_params=pltpu.CompilerParams(use_tc_tiling_on_sc=False),
    scratch_types=(
        [pltpu.VMEM((NUL, UL), jnp.int32)] * 2 +
        [pltpu.VMEM((NUF, UF), jnp.int32)] * 2 +
        [pltpu.VMEM((CV, 16), jnp.float32)] * 4 +
        [pltpu.VMEM((RL, D), jnp.float32)] * 2 +
        [pltpu.VMEM((RF, D), jnp.float32)] * 4 +
        [pltpu.VMEM((CV, D), jnp.float32)] * 6 +
        [pltpu.SemaphoreType.DMA] * 6
    ),
)
def _verts_kernel(y_hbm, ewf_hbm, nsf_hbm, lc_hbm, lv_hbm, fc_hbm, fv_hbm,
                  lap_hbm, gvew_hbm, gvns_hbm,
                  lidx0, lidx1, fidx0, fidx1, lval0, lval1, fval0, fval1,
                  lrows0, lrows1, erows0, erows1, nrows0, nrows1,
                  olap0, olap1, oew0, oew1, ons0, ons1,
                  s_st0, s_st1, s_rw0, s_rw1, s_out0, s_out1):
    wid = _wid()
    slot = (
        dict(lidx=lidx0, fidx=fidx0, lval=lval0, fval=fval0, lrows=lrows0,
             erows=erows0, nrows=nrows0, olap=olap0, oew=oew0, ons=ons0,
             s_st=s_st0, s_rw=s_rw0, s_out=s_out0),
        dict(lidx=lidx1, fidx=fidx1, lval=lval1, fval=fval1, lrows=lrows1,
             erows=erows1, nrows=nrows1, olap=olap1, oew=oew1, ons=ons1,
             s_st=s_st1, s_rw=s_rw1, s_out=s_out1),
    )

    def stage(c, s):
        d = slot[s]
        row0 = pl.multiple_of(wid * VW + c * CV, 8)
        lub = wid * (VW * 7 // UL) + c * NUL
        fub = wid * (VW * 6 // UF) + c * NUF
        pltpu.async_copy(lc_hbm.at[pl.ds(lub, NUL)], d["lidx"], d["s_st"])
        pltpu.async_copy(fc_hbm.at[pl.ds(fub, NUF)], d["fidx"], d["s_st"])
        pltpu.async_copy(lv_hbm.at[pl.ds(row0, CV)], d["lval"], d["s_st"])
        pltpu.async_copy(fv_hbm.at[pl.ds(row0, CV)], d["fval"], d["s_st"])

    def wait_stage(s):
        d = slot[s]
        pltpu.make_async_copy(lc_hbm.at[pl.ds(0, NUL)], d["lidx"],
                              d["s_st"]).wait()
        pltpu.make_async_copy(fc_hbm.at[pl.ds(0, NUF)], d["fidx"],
                              d["s_st"]).wait()
        pltpu.make_async_copy(lv_hbm.at[pl.ds(0, CV)], d["lval"],
                              d["s_st"]).wait()
        pltpu.make_async_copy(fv_hbm.at[pl.ds(0, CV)], d["fval"],
                              d["s_st"]).wait()

    def p1(c, s):
        d = slot[s]
        wait_stage(s)
        for u in range(NUL):
            pltpu.async_copy(y_hbm.at[d["lidx"].at[u]],
                             d["lrows"].at[pl.ds(u * UL, UL)], d["s_rw"])
        for u in range(NUF):
            pltpu.async_copy(ewf_hbm.at[d["fidx"].at[u]],
                             d["erows"].at[pl.ds(u * UF, UF)], d["s_rw"])
            pltpu.async_copy(nsf_hbm.at[d["fidx"].at[u]],
                             d["nrows"].at[pl.ds(u * UF, UF)], d["s_rw"])

    def wait_out(s):
        d = slot[s]
        pltpu.make_async_copy(d["olap"], lap_hbm.at[pl.ds(0, CV)],
                              d["s_out"]).wait()
        pltpu.make_async_copy(d["oew"], gvew_hbm.at[pl.ds(0, CV)],
                              d["s_out"]).wait()
        pltpu.make_async_copy(d["ons"], gvns_hbm.at[pl.ds(0, CV)],
                              d["s_out"]).wait()

    def p2(c, s, first=False, last=False):
        d = slot[s]
        for u in range(NUL):
            pltpu.make_async_copy(y_hbm.at[d["lidx"].at[u]],
                                  d["lrows"].at[pl.ds(u * UL, UL)],
                                  d["s_rw"]).wait()
        for u in range(NUF):
            pltpu.make_async_copy(ewf_hbm.at[d["fidx"].at[u]],
                                  d["erows"].at[pl.ds(u * UF, UF)],
                                  d["s_rw"]).wait()
            pltpu.make_async_copy(nsf_hbm.at[d["fidx"].at[u]],
                                  d["nrows"].at[pl.ds(u * UF, UF)],
                                  d["s_rw"]).wait()
        if not first:
            wait_out(s)
        lval_v, fval_v = d["lval"], d["fval"]
        lrows_v, erows_v, nrows_v = d["lrows"], d["erows"], d["nrows"]
        olap_v, oew_v, ons_v = d["olap"], d["oew"], d["ons"]

        def vert(i, carry):
            wl = lval_v[i]
            base = i * 7
            acc = [jnp.zeros((16,), jnp.float32) for _ in range(4)]
            for k in range(7):
                r = base + k
                wk = wl[k]
                for j in range(4):
                    acc[j] = acc[j] + wk * lrows_v[r, pl.ds(j * 16, 16)]
            for j in range(4):
                olap_v[i, pl.ds(j * 16, 16)] = acc[j]
            return carry

        lax.fori_loop(0, CV, vert, 0)

        def vert2(i, carry):
            wf = fval_v[i]
            base = i * 6
            acc = [jnp.zeros((16,), jnp.float32) for _ in range(8)]
            for k in range(6):
                r = base + k
                wk = wf[k]
                for j in range(4):
                    acc[j] = acc[j] + wk * erows_v[r, pl.ds(j * 16, 16)]
                    acc[4 + j] = acc[4 + j] + wk * nrows_v[r, pl.ds(j * 16, 16)]
            for j in range(4):
                oew_v[i, pl.ds(j * 16, 16)] = acc[j]
                ons_v[i, pl.ds(j * 16, 16)] = acc[4 + j]
            return carry

        lax.fori_loop(0, CV, vert2, 0)
        row0 = pl.multiple_of(wid * VW + c * CV, 8)
        pltpu.async_copy(olap_v, lap_hbm.at[pl.ds(row0, CV)], d["s_out"])
        pltpu.async_copy(oew_v, gvew_hbm.at[pl.ds(row0, CV)], d["s_out"])
        pltpu.async_copy(ons_v, gvns_hbm.at[pl.ds(row0, CV)], d["s_out"])
        if not last:
            stage(c + 2, s)

    stage(0, 0)
    stage(1, 1)
    p1(0, 0)
    p1(1, 1)
    p2(0, 0, first=True)
    p1(2, 0)
    p2(1, 1, first=True)

    def body(t, carry):
        c = 2 * t
        p1(c + 1, 1)
        p2(c, 0)
        p1(c + 2, 0)
        p2(c + 1, 1)
        return carry

    lax.fori_loop(1, NCHV // 2 - 1, body, 0)
    c = NCHV - 2
    p1(c + 1, 1)
    p2(c, 0, last=True)
    p2(c + 1, 1, last=True)
    wait_out(0)
    wait_out(1)


def _pad16(a):
    return jnp.pad(a, ((0, 0), (0, 16 - a.shape[1])))


_T = 512
_NT = (NV + _T - 1) // _T


def _transpose_block(x_ref, y_ref):
    y_ref[...] = x_ref[...].T


@functools.partial(jax.jit, static_argnums=())
def _make_y(x2d):
    return pl.pallas_call(
        _transpose_block,
        grid=(_NT,),
        in_specs=[pl.BlockSpec((D, _T), lambda i: (0, i))],
        out_specs=pl.BlockSpec((_T, D), lambda i: (i, 0)),
        out_shape=jax.ShapeDtypeStruct((NV, D), jnp.float32),
    )(x2d)


def _pack_block(x_ref, lap_ref, ew_ref, ns_ref, o_ref):
    o_ref[0] = x_ref[...]
    o_ref[1] = lap_ref[...].T.reshape(B, C, _T)
    o_ref[2] = ew_ref[...].T.reshape(B, C, _T)
    o_ref[3] = ns_ref[...].T.reshape(B, C, _T)


def _pack_out(x, lap, gvew, gvns):
    return pl.pallas_call(
        _pack_block,
        grid=(_NT,),
        in_specs=[
            pl.BlockSpec((B, C, _T), lambda i: (0, 0, i)),
            pl.BlockSpec((_T, D), lambda i: (i, 0)),
            pl.BlockSpec((_T, D), lambda i: (i, 0)),
            pl.BlockSpec((_T, D), lambda i: (i, 0)),
        ],
        out_specs=pl.BlockSpec((4, B, C, _T), lambda i: (0, 0, 0, i)),
        out_shape=jax.ShapeDtypeStruct((4, B, C, NV), jnp.float32),
    )(x, lap, gvew, gvns)


def kernel(x, G_vals, L_vals, F2V_vals, NS_dir, EW_dir, G_cols, L_cols,
           F2V_cols):
    y = _make_y(x.reshape(D, NV))
    gc9 = (G_cols.reshape(3, NF, 3).transpose(1, 0, 2)
           .reshape(NF * 9 // UA, UA))
    gv9 = _pad16(G_vals.reshape(3, NF, 3).transpose(1, 0, 2).reshape(NF, 9))
    ew9 = _pad16(jnp.repeat(EW_dir, 3, axis=1))
    ns9 = _pad16(jnp.repeat(NS_dir, 3, axis=1))

    pad = NVP - NV
    lc = jnp.pad(L_cols, ((0, pad), (0, 0))).reshape(NVP * 7 // UL, UL)
    lv = _pad16(jnp.pad(L_vals, ((0, pad), (0, 0))))
    fc = jnp.pad(F2V_cols, ((0, pad), (0, 0))).reshape(NVP * 6 // UF, UF)
    fv = _pad16(jnp.pad(F2V_vals, ((0, pad), (0, 0))))

    ewf, nsf = _faces_kernel(y, gc9, gv9, ew9, ns9)
    lap, gvew, gvns = _verts_kernel(y, ewf, nsf, lc, lv, fc, fv)
    return _pack_out(x, lap, gvew, gvns)


# speedup vs baseline: 1.0610x; 1.0610x over previous
import functools
import jax
import jax.numpy as jnp
from jax import lax
from jax.experimental import pallas as pl
from jax.experimental.pallas import tpu as pltpu
from jax.experimental.pallas import tpu_sc as plsc

NV = 40962
NF = 81920
B = 2
C = 32
D = B * C

NC = 2
NS = 16
NW = NC * NS

FW = NF // NW
CF = 64
NCHA = FW // CF
RA = CF * 9
GA = CF * 3
UA = 96
NUA = RA // UA

NVP = 43008
VW = NVP // NW
CV = 32
NCHV = VW // CV
RL = CV * 7
UL = 112
NUL = RL // UL
RF = CV * 6
UF = 96
NUF = RF // UF

_mesh = plsc.VectorSubcoreMesh(core_axis_name="c", subcore_axis_name="s")


def _wid():
    return lax.axis_index("s") * NC + lax.axis_index("c")


@functools.partial(
    pl.kernel,
    out_type=[
        jax.ShapeDtypeStruct((NF, D), jnp.float32),
        jax.ShapeDtypeStruct((NF, D), jnp.float32),
    ],
    mesh=_mesh,
    compiler_params=pltpu.CompilerParams(use_tc_tiling_on_sc=False),
    scratch_types=(
        [pltpu.VMEM((RA,), jnp.int32)] * 2 +
        [pltpu.VMEM((RA + 16,), jnp.float32)] * 2 +
        [pltpu.VMEM((GA + 16,), jnp.float32)] * 4 +
        [pltpu.VMEM((RA, D), jnp.float32)] * 2 +
        [pltpu.VMEM((CF, D), jnp.float32)] * 4 +
        [pltpu.SemaphoreType.DMA] * 6
    ),
)
def _faces_kernel(y_hbm, gc_hbm, gv_hbm, ew_hbm, ns_hbm, ewf_hbm, nsf_hbm,
                  idx0, idx1, gv0, gv1, ew0, ew1, ns0, ns1,
                  rows0, rows1, oew0, oew1, ons0, ons1,
                  s_st0, s_st1, s_rw0, s_rw1, s_out0, s_out1):
    wid = _wid()
    slot = (
        dict(idx=idx0, gv=gv0, ew=ew0, ns=ns0, rows=rows0, oew=oew0,
             ons=ons0, s_st=s_st0, s_rw=s_rw0, s_out=s_out0),
        dict(idx=idx1, gv=gv1, ew=ew1, ns=ns1, rows=rows1, oew=oew1,
             ons=ons1, s_st=s_st1, s_rw=s_rw1, s_out=s_out1),
    )

    def stage(c, s):
        d = slot[s]
        foff = pl.multiple_of((wid * FW + c * CF) * 3, 8)
        for g in range(3):
            src = pl.multiple_of(g * (NF * 3), 8) + foff
            pltpu.async_copy(gc_hbm.at[pl.ds(src, GA)],
                             d["idx"].at[pl.ds(g * GA, GA)], d["s_st"])
            pltpu.async_copy(gv_hbm.at[pl.ds(src, GA)],
                             d["gv"].at[pl.ds(g * GA, GA)], d["s_st"])
        pltpu.async_copy(ew_hbm.at[pl.ds(foff, GA)],
                         d["ew"].at[pl.ds(0, GA)], d["s_st"])
        pltpu.async_copy(ns_hbm.at[pl.ds(foff, GA)],
                         d["ns"].at[pl.ds(0, GA)], d["s_st"])

    def wait_stage(s):
        d = slot[s]
        for g in range(3):
            pltpu.make_async_copy(gc_hbm.at[pl.ds(0, GA)],
                                  d["idx"].at[pl.ds(g * GA, GA)],
                                  d["s_st"]).wait()
            pltpu.make_async_copy(gv_hbm.at[pl.ds(0, GA)],
                                  d["gv"].at[pl.ds(g * GA, GA)],
                                  d["s_st"]).wait()
        pltpu.make_async_copy(ew_hbm.at[pl.ds(0, GA)],
                              d["ew"].at[pl.ds(0, GA)], d["s_st"]).wait()
        pltpu.make_async_copy(ns_hbm.at[pl.ds(0, GA)],
                              d["ns"].at[pl.ds(0, GA)], d["s_st"]).wait()

    def p1(c, s):
        d = slot[s]
        wait_stage(s)
        for u in range(NUA):
            pltpu.async_copy(y_hbm.at[d["idx"].at[pl.ds(u * UA, UA)]],
                             d["rows"].at[pl.ds(u * UA, UA)], d["s_rw"])

    def wait_out(s):
        d = slot[s]
        pltpu.make_async_copy(d["oew"], ewf_hbm.at[pl.ds(0, CF)],
                              d["s_out"]).wait()
        pltpu.make_async_copy(d["ons"], nsf_hbm.at[pl.ds(0, CF)],
                              d["s_out"]).wait()

    def p2(c, s, first=False, last=False):
        d = slot[s]
        for u in range(NUA):
            pltpu.make_async_copy(y_hbm.at[d["idx"].at[pl.ds(u * UA, UA)]],
                                  d["rows"].at[pl.ds(u * UA, UA)],
                                  d["s_rw"]).wait()
        if not first:
            wait_out(s)
        rows_v, oew_v, ons_v = d["rows"], d["oew"], d["ons"]
        gv_v, ew_v, ns_v = d["gv"], d["ew"], d["ns"]

        def face(i, carry):
            i3 = i * 3
            ewv = ew_v[pl.ds(i3, 16)]
            nsv = ns_v[pl.ds(i3, 16)]
            acc = [jnp.zeros((16,), jnp.float32) for _ in range(8)]
            for g in range(3):
                gvv = gv_v[pl.ds(g * GA + i3, 16)]
                eg = ewv[g]
                ng = nsv[g]
                for k in range(3):
                    we = eg * gvv[k]
                    wn = ng * gvv[k]
                    r = g * GA + i3 + k
                    for j in range(4):
                        row = rows_v[r, pl.ds(j * 16, 16)]
                        acc[j] = acc[j] + we * row
                        acc[4 + j] = acc[4 + j] + wn * row
            for j in range(4):
                oew_v[i, pl.ds(j * 16, 16)] = acc[j]
                ons_v[i, pl.ds(j * 16, 16)] = acc[4 + j]
            return carry

        lax.fori_loop(0, CF, face, 0)
        if not last:
            stage(c + 2, s)
        face0 = pl.multiple_of(wid * FW + c * CF, 8)
        pltpu.async_copy(oew_v, ewf_hbm.at[pl.ds(face0, CF)], d["s_out"])
        pltpu.async_copy(ons_v, nsf_hbm.at[pl.ds(face0, CF)], d["s_out"])

    stage(0, 0)
    stage(1, 1)
    p1(0, 0)
    p1(1, 1)
    p2(0, 0, first=True)
    p1(2, 0)
    p2(1, 1, first=True)

    def body(t, carry):
        c = 2 * t
        p1(c + 1, 1)
        p2(c, 0)
        p1(c + 2, 0)
        p2(c + 1, 1)
        return carry

    lax.fori_loop(1, NCHA // 2 - 1, body, 0)
    c = NCHA - 2
    p1(c + 1, 1)
    p2(c, 0, last=True)
    p2(c + 1, 1, last=True)
    wait_out(0)
    wait_out(1)


@functools.partial(
    pl.kernel,
    out_type=[
        jax.ShapeDtypeStruct((NVP, D), jnp.float32),
        jax.ShapeDtypeStruct((NVP, D), jnp.float32),
        jax.ShapeDtypeStruct((NVP, D), jnp.float32),
    ],
    mesh=_mesh,
    compiler_params=pltpu.CompilerParams(use_tc_tiling_on_sc=False),
    scratch_types=(
        [pltpu.VMEM((RL,), jnp.int32)] * 2 +
        [pltpu.VMEM((RF,), jnp.int32)] * 2 +
        [pltpu.VMEM((RL + 16,), jnp.float32)] * 2 +
        [pltpu.VMEM((RF + 16,), jnp.float32)] * 2 +
        [pltpu.VMEM((RL, D), jnp.float32)] * 2 +
        [pltpu.VMEM((RF, D), jnp.float32)] * 4 +
        [pltpu.VMEM((CV, D), jnp.float32)] * 6 +
        [pltpu.SemaphoreType.DMA] * 6
    ),
)
def _verts_kernel(y_hbm, ewf_hbm, nsf_hbm, lc_hbm, lv_hbm, fc_hbm, fv_hbm,
                  lap_hbm, gvew_hbm, gvns_hbm,
                  lidx0, lidx1, fidx0, fidx1, lval0, lval1, fval0, fval1,
                  lrows0, lrows1, erows0, erows1, nrows0, nrows1,
                  olap0, olap1, oew0, oew1, ons0, ons1,
                  s_st0, s_st1, s_rw0, s_rw1, s_out0, s_out1):
    wid = _wid()
    slot = (
        dict(lidx=lidx0, fidx=fidx0, lval=lval0, fval=fval0, lrows=lrows0,
             erows=erows0, nrows=nrows0, olap=olap0, oew=oew0, ons=ons0,
             s_st=s_st0, s_rw=s_rw0, s_out=s_out0),
        dict(lidx=lidx1, fidx=fidx1, lval=lval1, fval=fval1, lrows=lrows1,
             erows=erows1, nrows=nrows1, olap=olap1, oew=oew1, ons=ons1,
             s_st=s_st1, s_rw=s_rw1, s_out=s_out1),
    )

    def stage(c, s):
        d = slot[s]
        loff = pl.multiple_of((wid * VW + c * CV) * 7, 8)
        foff = pl.multiple_of((wid * VW + c * CV) * 6, 8)
        pltpu.async_copy(lc_hbm.at[pl.ds(loff, RL)],
                         d["lidx"], d["s_st"])
        pltpu.async_copy(fc_hbm.at[pl.ds(foff, RF)],
                         d["fidx"], d["s_st"])
        pltpu.async_copy(lv_hbm.at[pl.ds(loff, RL)],
                         d["lval"].at[pl.ds(0, RL)], d["s_st"])
        pltpu.async_copy(fv_hbm.at[pl.ds(foff, RF)],
                         d["fval"].at[pl.ds(0, RF)], d["s_st"])

    def wait_stage(s):
        d = slot[s]
        pltpu.make_async_copy(lc_hbm.at[pl.ds(0, RL)], d["lidx"],
                              d["s_st"]).wait()
        pltpu.make_async_copy(fc_hbm.at[pl.ds(0, RF)], d["fidx"],
                              d["s_st"]).wait()
        pltpu.make_async_copy(lv_hbm.at[pl.ds(0, RL)],
                              d["lval"].at[pl.ds(0, RL)], d["s_st"]).wait()
        pltpu.make_async_copy(fv_hbm.at[pl.ds(0, RF)],
                              d["fval"].at[pl.ds(0, RF)], d["s_st"]).wait()

    def p1(c, s):
        d = slot[s]
        wait_stage(s)
        for u in range(NUL):
            pltpu.async_copy(y_hbm.at[d["lidx"].at[pl.ds(u * UL, UL)]],
                             d["lrows"].at[pl.ds(u * UL, UL)], d["s_rw"])
        for u in range(NUF):
            pltpu.async_copy(ewf_hbm.at[d["fidx"].at[pl.ds(u * UF, UF)]],
                             d["erows"].at[pl.ds(u * UF, UF)], d["s_rw"])
            pltpu.async_copy(nsf_hbm.at[d["fidx"].at[pl.ds(u * UF, UF)]],
                             d["nrows"].at[pl.ds(u * UF, UF)], d["s_rw"])

    def wait_out(s):
        d = slot[s]
        pltpu.make_async_copy(d["olap"], lap_hbm.at[pl.ds(0, CV)],
                              d["s_out"]).wait()
        pltpu.make_async_copy(d["oew"], gvew_hbm.at[pl.ds(0, CV)],
                              d["s_out"]).wait()
        pltpu.make_async_copy(d["ons"], gvns_hbm.at[pl.ds(0, CV)],
                              d["s_out"]).wait()

    def p2(c, s, first=False, last=False):
        d = slot[s]
        for u in range(NUL):
            pltpu.make_async_copy(y_hbm.at[d["lidx"].at[pl.ds(u * UL, UL)]],
                                  d["lrows"].at[pl.ds(u * UL, UL)],
                                  d["s_rw"]).wait()
        for u in range(NUF):
            pltpu.make_async_copy(ewf_hbm.at[d["fidx"].at[pl.ds(u * UF, UF)]],
                                  d["erows"].at[pl.ds(u * UF, UF)],
                                  d["s_rw"]).wait()
            pltpu.make_async_copy(nsf_hbm.at[d["fidx"].at[pl.ds(u * UF, UF)]],
                                  d["nrows"].at[pl.ds(u * UF, UF)],
                                  d["s_rw"]).wait()
        if not first:
            wait_out(s)
        lval_v, fval_v = d["lval"], d["fval"]
        lrows_v, erows_v, nrows_v = d["lrows"], d["erows"], d["nrows"]
        olap_v, oew_v, ons_v = d["olap"], d["oew"], d["ons"]

        def vert(i, carry):
            base = i * 7
            wl = lval_v[pl.ds(base, 16)]
            acc = [jnp.zeros((16,), jnp.float32) for _ in range(4)]
            for k in range(7):
                wk = wl[k]
                r = base + k
                for j in range(4):
                    acc[j] = acc[j] + wk * lrows_v[r, pl.ds(j * 16, 16)]
            for j in range(4):
                olap_v[i, pl.ds(j * 16, 16)] = acc[j]
            return carry

        lax.fori_loop(0, CV, vert, 0)

        def vert2(i, carry):
            base = i * 6
            wf = fval_v[pl.ds(base, 16)]
            acc = [jnp.zeros((16,), jnp.float32) for _ in range(8)]
            for k in range(6):
                wk = wf[k]
                r = base + k
                for j in range(4):
                    acc[j] = acc[j] + wk * erows_v[r, pl.ds(j * 16, 16)]
                    acc[4 + j] = acc[4 + j] + wk * nrows_v[r, pl.ds(j * 16, 16)]
            for j in range(4):
                oew_v[i, pl.ds(j * 16, 16)] = acc[j]
                ons_v[i, pl.ds(j * 16, 16)] = acc[4 + j]
            return carry

        lax.fori_loop(0, CV, vert2, 0)
        if not last:
            stage(c + 2, s)
        row0 = pl.multiple_of(wid * VW + c * CV, 8)
        pltpu.async_copy(olap_v, lap_hbm.at[pl.ds(row0, CV)], d["s_out"])
        pltpu.async_copy(oew_v, gvew_hbm.at[pl.ds(row0, CV)], d["s_out"])
        pltpu.async_copy(ons_v, gvns_hbm.at[pl.ds(row0, CV)], d["s_out"])

    stage(0, 0)
    stage(1, 1)
    p1(0, 0)
    p1(1, 1)
    p2(0, 0, first=True)
    p1(2, 0)
    p2(1, 1, first=True)

    def body(t, carry):
        c = 2 * t
        p1(c + 1, 1)
        p2(c, 0)
        p1(c + 2, 0)
        p2(c + 1, 1)
        return carry

    lax.fori_loop(1, NCHV // 2 - 1, body, 0)
    c = NCHV - 2
    p1(c + 1, 1)
    p2(c, 0, last=True)
    p2(c + 1, 1, last=True)
    wait_out(0)
    wait_out(1)


_T = 512
_NT = (NV + _T - 1) // _T


def _transpose_block(x_ref, y_ref):
    y_ref[...] = x_ref[...].T


def _make_y(x2d):
    return pl.pallas_call(
        _transpose_block,
        grid=(_NT,),
        in_specs=[pl.BlockSpec((D, _T), lambda i: (0, i))],
        out_specs=pl.BlockSpec((_T, D), lambda i: (i, 0)),
        out_shape=jax.ShapeDtypeStruct((NV, D), jnp.float32),
    )(x2d)


def _pack_block(x_ref, lap_ref, ew_ref, ns_ref, o_ref):
    o_ref[0] = x_ref[...]
    o_ref[1] = lap_ref[...].T.reshape(B, C, _T)
    o_ref[2] = ew_ref[...].T.reshape(B, C, _T)
    o_ref[3] = ns_ref[...].T.reshape(B, C, _T)


def _pack_out(x, lap, gvew, gvns):
    return pl.pallas_call(
        _pack_block,
        grid=(_NT,),
        in_specs=[
            pl.BlockSpec((B, C, _T), lambda i: (0, 0, i)),
            pl.BlockSpec((_T, D), lambda i: (i, 0)),
            pl.BlockSpec((_T, D), lambda i: (i, 0)),
            pl.BlockSpec((_T, D), lambda i: (i, 0)),
        ],
        out_specs=pl.BlockSpec((4, B, C, _T), lambda i: (0, 0, 0, i)),
        out_shape=jax.ShapeDtypeStruct((4, B, C, NV), jnp.float32),
    )(x, lap, gvew, gvns)


def kernel(x, G_vals, L_vals, F2V_vals, NS_dir, EW_dir, G_cols, L_cols,
           F2V_cols):
    y = _make_y(x.reshape(D, NV))
    gcf = G_cols.reshape(3 * NF * 3)
    gvf = G_vals.reshape(3 * NF * 3)
    ewfl = EW_dir.reshape(NF * 3)
    nsfl = NS_dir.reshape(NF * 3)

    pad = NVP - NV
    lc = jnp.pad(L_cols, ((0, pad), (0, 0))).reshape(NVP * 7)
    lv = jnp.pad(L_vals, ((0, pad), (0, 0))).reshape(NVP * 7)
    fc = jnp.pad(F2V_cols, ((0, pad), (0, 0))).reshape(NVP * 6)
    fv = jnp.pad(F2V_vals, ((0, pad), (0, 0))).reshape(NVP * 6)

    ewf, nsf = _faces_kernel(y, gcf, gvf, ewfl, nsfl)
    lap, gvew, gvns = _verts_kernel(y, ewf, nsf, lc, lv, fc, fv)
    return _pack_out(x, lap, gvew, gvns)
